# baseline (speedup 1.0000x reference)
import jax
import jax.numpy as jnp
from jax.experimental import pallas as pl

_N_GROUP = 2
_GROUP_SIZE = 4
_ROUTED_SCALING = 2.5

_BLOCK_N = 1024


def _gate_kernel(x_ref, w_ref, logits_ref, idx_ref, wgt_ref):
    x = x_ref[...]
    w = w_ref[...]
    logits = jax.lax.dot_general(
        x, w, (((1,), (1,)), ((), ())), preferred_element_type=jnp.float32
    )
    logits_ref[...] = logits
    scores = jax.nn.sigmoid(logits)

    col = jax.lax.broadcasted_iota(jnp.int32, scores.shape, 1)
    in_g0 = col < _GROUP_SIZE
    neg = jnp.float32(-jnp.inf)
    m0 = jnp.max(jnp.where(in_g0, scores, neg), axis=1, keepdims=True)
    m1 = jnp.max(jnp.where(in_g0, neg, scores), axis=1, keepdims=True)
    big = jnp.int32(_N_GROUP * _GROUP_SIZE)
    i0 = jnp.min(jnp.where(in_g0 & (scores >= m0), col, big),
                 axis=1, keepdims=True)
    i1 = jnp.min(jnp.where((~in_g0) & (scores >= m1), col, big),
                 axis=1, keepdims=True)
    inv = _ROUTED_SCALING / (m0 + m1 + 1e-10)
    idx_ref[...] = jnp.concatenate([i0, i1], axis=1)
    wgt_ref[...] = jnp.concatenate([m0 * inv, m1 * inv], axis=1)


def kernel(hidden_states, gate_weight):
    n, d = hidden_states.shape
    e = gate_weight.shape[0]
    gate_logits, topk_idx, topk_weight = pl.pallas_call(
        _gate_kernel,
        grid=(n // _BLOCK_N,),
        in_specs=[
            pl.BlockSpec((_BLOCK_N, d), lambda i: (i, 0)),
            pl.BlockSpec((e, d), lambda i: (0, 0)),
        ],
        out_specs=[
            pl.BlockSpec((_BLOCK_N, e), lambda i: (i, 0)),
            pl.BlockSpec((_BLOCK_N, _N_GROUP), lambda i: (i, 0)),
            pl.BlockSpec((_BLOCK_N, _N_GROUP), lambda i: (i, 0)),
        ],
        out_shape=[
            jax.ShapeDtypeStruct((n, e), jnp.float32),
            jax.ShapeDtypeStruct((n, _N_GROUP), jnp.int32),
            jax.ShapeDtypeStruct((n, _N_GROUP), jnp.float32),
        ],
    )(hidden_states, gate_weight)
    return (topk_idx, topk_weight, gate_logits)
